# Initial kernel scaffold; baseline (speedup 1.0000x reference)
import functools

import jax
import jax.numpy as jnp
from jax import lax
from jax.experimental import pallas as pl
from jax.experimental.pallas import tpu as pltpu
from jax.experimental.pallas import tpu_sc as plsc

NC = 2
NS = 16
LANES = 16
NWORK = NC * NS
CHUNK = 128


def _sc_kernel(n_total, vocab, embed, n_neigh, seq):
    pairs = embed // 2
    n_w = n_total // NWORK
    nchunks = n_w // CHUNK
    mesh = plsc.VectorSubcoreMesh(
        core_axis_name="c", subcore_axis_name="s",
        num_cores=NC, num_subcores=NS)

    @functools.partial(
        pl.kernel,
        out_type=jax.ShapeDtypeStruct((n_total,), jnp.float32),
        mesh=mesh,
        scratch_types=dict(
            idx_g=pltpu.VMEM((CHUNK,), jnp.int32),
            idx_c=pltpu.VMEM((CHUNK,), jnp.int32),
            wrows=pltpu.VMEM((CHUNK, pairs), jnp.int32),
            crows=pltpu.VMEM((CHUNK, pairs), jnp.int32),
            outv=pltpu.VMEM((CHUNK,), jnp.float32),
            pw_v=pltpu.VMEM((n_neigh,), jnp.float32),
            bias_v=pltpu.VMEM((vocab,), jnp.float32),
            sem_w=pltpu.SemaphoreType.DMA,
            sem_c=pltpu.SemaphoreType.DMA,
        ),
    )
    def run(gram_hbm, ctx_hbm, wt_hbm, ct_hbm, pw_hbm, bias_hbm, out_hbm,
            idx_g, idx_c, wrows, crows, outv, pw_v, bias_v, sem_w, sem_c):
        wid = lax.axis_index("s") * NC + lax.axis_index("c")
        base = wid * n_w
        pltpu.sync_copy(bias_hbm, bias_v)
        pltpu.sync_copy(pw_hbm, pw_v)
        iota16 = lax.iota(jnp.int32, LANES)

        @pl.loop(0, nchunks)
        def chunk_body(i):
            start = base + i * CHUNK
            pltpu.sync_copy(gram_hbm.at[pl.ds(start, CHUNK)], idx_g)
            pltpu.sync_copy(ctx_hbm.at[pl.ds(start, CHUNK)], idx_c)
            cw = pltpu.async_copy(wt_hbm.at[idx_g], wrows, sem_w)
            cc = pltpu.async_copy(ct_hbm.at[idx_c], crows, sem_c)
            cw.wait()
            cc.wait()
            for g in range(CHUNK // LANES):
                rowv = iota16 + (g * LANES)
                acc_a = jnp.zeros((LANES,), jnp.float32)
                acc_b = jnp.zeros((LANES,), jnp.float32)
                for h in range(pairs):
                    colv = jnp.full((LANES,), h, jnp.int32)
                    wv = plsc.load_gather(wrows, [rowv, colv])
                    cv = plsc.load_gather(crows, [rowv, colv])
                    wb = plsc.bitcast(wv, jnp.bfloat16)
                    cb = plsc.bitcast(cv, jnp.bfloat16)
                    pa, pb = plsc.unpack(wb * cb, format=plsc.PackFormat.INTERLEAVED)
                    acc_a = acc_a + pa
                    acc_b = acc_b + pb
                posv = iota16 + (start + g * LANES)
                xv = (posv // seq) % n_neigh
                pwv = plsc.load_gather(pw_v, [xv])
                idv = idx_c[pl.ds(g * LANES, LANES)]
                bv = plsc.load_gather(bias_v, [idv])
                outv[pl.ds(g * LANES, LANES)] = (acc_a + acc_b) * pwv + bv
            pltpu.sync_copy(outv, out_hbm.at[pl.ds(start, CHUNK)])

    return run


def kernel(gram, context, word_table, context_table, position_weights, bias):
    b, n_neigh, seq = gram.shape
    vocab, embed = word_table.shape
    n_total = b * n_neigh * seq
    assert n_total % (NWORK * CHUNK) == 0

    gram_f = gram.reshape(n_total).astype(jnp.int32)
    ctx_f = context.reshape(n_total).astype(jnp.int32)
    wt_p = lax.bitcast_convert_type(
        word_table.astype(jnp.bfloat16).reshape(vocab, embed // 2, 2), jnp.int32)
    ct_p = lax.bitcast_convert_type(
        context_table.astype(jnp.bfloat16).reshape(vocab, embed // 2, 2), jnp.int32)

    run = _sc_kernel(n_total, vocab, embed, n_neigh, seq)
    out_flat = run(gram_f, ctx_f, wt_p, ct_p,
                   position_weights.astype(jnp.float32), bias.astype(jnp.float32))
    return out_flat.reshape(b, n_neigh, seq)


# speedup vs baseline: 10.0016x; 10.0016x over previous
import functools

import jax
import jax.numpy as jnp
from jax import lax
from jax.experimental import pallas as pl
from jax.experimental.pallas import tpu as pltpu
from jax.experimental.pallas import tpu_sc as plsc

NC = 2
NS = 16
LANES = 16
NWORK = NC * NS
CHUNK = 128


def _sc_kernel(n_total, vocab, embed, n_neigh, seq):
    pairs = embed // 2
    n_w = n_total // NWORK
    nchunks = n_w // CHUNK
    mesh = plsc.VectorSubcoreMesh(
        core_axis_name="c", subcore_axis_name="s",
        num_cores=NC, num_subcores=NS)

    @functools.partial(
        pl.kernel,
        out_type=jax.ShapeDtypeStruct((n_total,), jnp.float32),
        mesh=mesh,
        compiler_params=pltpu.CompilerParams(
            needs_layout_passes=False, use_tc_tiling_on_sc=False),
        scratch_types=dict(
            idx_g=pltpu.VMEM((CHUNK,), jnp.int32),
            idx_c=pltpu.VMEM((CHUNK,), jnp.int32),
            wrows=pltpu.VMEM((CHUNK, pairs), jnp.int32),
            crows=pltpu.VMEM((CHUNK, pairs), jnp.int32),
            outv=pltpu.VMEM((CHUNK,), jnp.float32),
            pw_v=pltpu.VMEM((n_neigh,), jnp.float32),
            bias_v=pltpu.VMEM((vocab,), jnp.float32),
            sem_w=pltpu.SemaphoreType.DMA,
            sem_c=pltpu.SemaphoreType.DMA,
        ),
    )
    def run(gram_hbm, ctx_hbm, wt_hbm, ct_hbm, pw_hbm, bias_hbm, out_hbm,
            idx_g, idx_c, wrows, crows, outv, pw_v, bias_v, sem_w, sem_c):
        wid = lax.axis_index("s") * NC + lax.axis_index("c")
        base = wid * n_w
        pltpu.sync_copy(bias_hbm, bias_v)
        pltpu.sync_copy(pw_hbm, pw_v)
        iota16 = lax.iota(jnp.int32, LANES)

        @pl.loop(0, nchunks)
        def chunk_body(i):
            start = base + i * CHUNK
            pltpu.sync_copy(gram_hbm.at[pl.ds(start, CHUNK)], idx_g)
            pltpu.sync_copy(ctx_hbm.at[pl.ds(start, CHUNK)], idx_c)
            cw = pltpu.async_copy(wt_hbm.at[idx_g], wrows, sem_w)
            cc = pltpu.async_copy(ct_hbm.at[idx_c], crows, sem_c)
            cw.wait()
            cc.wait()
            for g in range(CHUNK // LANES):
                rowv = iota16 + (g * LANES)
                acc_a = jnp.zeros((LANES,), jnp.float32)
                acc_b = jnp.zeros((LANES,), jnp.float32)
                for h in range(pairs):
                    colv = jnp.full((LANES,), h, jnp.int32)
                    wv = plsc.load_gather(wrows, [rowv, colv])
                    cv = plsc.load_gather(crows, [rowv, colv])
                    wb = plsc.bitcast(wv, jnp.bfloat16)
                    cb = plsc.bitcast(cv, jnp.bfloat16)
                    pa, pb = plsc.unpack(wb * cb, format=plsc.PackFormat.INTERLEAVED)
                    acc_a = acc_a + pa
                    acc_b = acc_b + pb
                posv = iota16 + (start + g * LANES)
                xv = (posv // seq) % n_neigh
                pwv = plsc.load_gather(pw_v, [xv])
                idv = idx_c[pl.ds(g * LANES, LANES)]
                bv = plsc.load_gather(bias_v, [idv])
                outv[pl.ds(g * LANES, LANES)] = (acc_a + acc_b) * pwv + bv
            pltpu.sync_copy(outv, out_hbm.at[pl.ds(start, CHUNK)])

    return run


def kernel(gram, context, word_table, context_table, position_weights, bias):
    b, n_neigh, seq = gram.shape
    vocab, embed = word_table.shape
    n_total = b * n_neigh * seq
    assert n_total % (NWORK * CHUNK) == 0

    gram_f = gram.reshape(n_total).astype(jnp.int32)
    ctx_f = context.reshape(n_total).astype(jnp.int32)
    wt_p = lax.bitcast_convert_type(
        word_table.astype(jnp.bfloat16).reshape(vocab, embed // 2, 2), jnp.int32)
    ct_p = lax.bitcast_convert_type(
        context_table.astype(jnp.bfloat16).reshape(vocab, embed // 2, 2), jnp.int32)

    run = _sc_kernel(n_total, vocab, embed, n_neigh, seq)
    out_flat = run(gram_f, ctx_f, wt_p, ct_p,
                   position_weights.astype(jnp.float32), bias.astype(jnp.float32))
    return out_flat.reshape(b, n_neigh, seq)


# speedup vs baseline: 11.7477x; 1.1746x over previous
import functools

import jax
import jax.numpy as jnp
from jax import lax
from jax.experimental import pallas as pl
from jax.experimental.pallas import tpu as pltpu
from jax.experimental.pallas import tpu_sc as plsc

NC = 2
NS = 16
LANES = 16
NWORK = NC * NS
CHUNK = 128


def _sc_kernel(n_total, vocab, embed, n_neigh, seq):
    pairs = embed // 2
    n_w = n_total // NWORK
    nchunks = n_w // CHUNK
    assert nchunks % 2 == 0
    groups = CHUNK // LANES
    mesh = plsc.VectorSubcoreMesh(
        core_axis_name="c", subcore_axis_name="s",
        num_cores=NC, num_subcores=NS)

    @functools.partial(
        pl.kernel,
        out_type=jax.ShapeDtypeStruct((n_total,), jnp.float32),
        mesh=mesh,
        compiler_params=pltpu.CompilerParams(
            needs_layout_passes=False, use_tc_tiling_on_sc=False),
        scratch_types=dict(
            idx_g=pltpu.VMEM((2, CHUNK), jnp.int32),
            idx_c=pltpu.VMEM((2, CHUNK), jnp.int32),
            wrows=pltpu.VMEM((2, CHUNK, pairs), jnp.int32),
            crows=pltpu.VMEM((2, CHUNK, pairs), jnp.int32),
            bv=pltpu.VMEM((2, CHUNK), jnp.float32),
            outv=pltpu.VMEM((2, CHUNK), jnp.float32),
            pw_v=pltpu.VMEM((n_neigh,), jnp.float32),
            bias_v=pltpu.VMEM((vocab,), jnp.float32),
            sem_idx=pltpu.SemaphoreType.DMA((2,)),
            sem_rows=pltpu.SemaphoreType.DMA((2,)),
            sem_out=pltpu.SemaphoreType.DMA((2,)),
        ),
    )
    def run(gram_hbm, ctx_hbm, wt_hbm, ct_hbm, pw_hbm, bias_hbm, out_hbm,
            idx_g, idx_c, wrows, crows, bv, outv, pw_v, bias_v,
            sem_idx, sem_rows, sem_out):
        wid = lax.axis_index("s") * NC + lax.axis_index("c")
        base = wid * n_w
        pltpu.sync_copy(bias_hbm, bias_v)
        pltpu.sync_copy(pw_hbm, pw_v)
        iota16 = lax.iota(jnp.int32, LANES)

        def fire_idx(c, slot):
            st = base + c * CHUNK
            pltpu.async_copy(gram_hbm.at[pl.ds(st, CHUNK)], idx_g.at[slot],
                             sem_idx.at[slot])
            pltpu.async_copy(ctx_hbm.at[pl.ds(st, CHUNK)], idx_c.at[slot],
                             sem_idx.at[slot])

        def wait_idx(c, slot):
            st = base + c * CHUNK
            pltpu.make_async_copy(gram_hbm.at[pl.ds(st, CHUNK)],
                                  idx_g.at[slot], sem_idx.at[slot]).wait()
            pltpu.make_async_copy(ctx_hbm.at[pl.ds(st, CHUNK)],
                                  idx_c.at[slot], sem_idx.at[slot]).wait()

        def fire_rows(slot):
            pltpu.async_copy(wt_hbm.at[idx_g.at[slot]], wrows.at[slot],
                             sem_rows.at[slot])
            pltpu.async_copy(ct_hbm.at[idx_c.at[slot]], crows.at[slot],
                             sem_rows.at[slot])

        def wait_rows(slot):
            pltpu.make_async_copy(wt_hbm.at[idx_g.at[slot]], wrows.at[slot],
                                  sem_rows.at[slot]).wait()
            pltpu.make_async_copy(ct_hbm.at[idx_c.at[slot]], crows.at[slot],
                                  sem_rows.at[slot]).wait()

        def prefetch_bias(slot):
            for g in range(groups):
                idv = idx_c[slot, pl.ds(g * LANES, LANES)]
                bv[slot, pl.ds(g * LANES, LANES)] = plsc.load_gather(
                    bias_v, [idv])

        def fire_out(c, slot):
            st = base + c * CHUNK
            pltpu.async_copy(outv.at[slot], out_hbm.at[pl.ds(st, CHUNK)],
                             sem_out.at[slot])

        def wait_out(c, slot):
            st = base + c * CHUNK
            pltpu.make_async_copy(outv.at[slot], out_hbm.at[pl.ds(st, CHUNK)],
                                  sem_out.at[slot]).wait()

        def compute(c, slot):
            wr = wrows.at[slot]
            cr = crows.at[slot]
            st = base + c * CHUNK
            for g in range(groups):
                rowv = iota16 + (g * LANES)
                acc_a = jnp.zeros((LANES,), jnp.float32)
                acc_b = jnp.zeros((LANES,), jnp.float32)
                for h in range(pairs):
                    colv = jnp.full((LANES,), h, jnp.int32)
                    wv = plsc.load_gather(wr, [rowv, colv])
                    cv = plsc.load_gather(cr, [rowv, colv])
                    wb = plsc.bitcast(wv, jnp.bfloat16)
                    cb = plsc.bitcast(cv, jnp.bfloat16)
                    pa, pb = plsc.unpack(wb * cb,
                                         format=plsc.PackFormat.INTERLEAVED)
                    acc_a = acc_a + pa
                    acc_b = acc_b + pb
                posv = iota16 + (st + g * LANES)
                xv = (posv // seq) % n_neigh
                pwv = plsc.load_gather(pw_v, [xv])
                bvv = bv[slot, pl.ds(g * LANES, LANES)]
                outv[slot, pl.ds(g * LANES, LANES)] = \
                    (acc_a + acc_b) * pwv + bvv

        fire_idx(0, 0)
        fire_idx(1, 1)
        wait_idx(0, 0)
        fire_rows(0)
        prefetch_bias(0)

        @pl.loop(0, nchunks // 2)
        def chunk_pair(j):
            for b in range(2):
                c = 2 * j + b
                slot = b
                nxt = 1 - b

                @pl.when(c + 1 < nchunks)
                def _():
                    wait_idx(c + 1, nxt)
                    fire_rows(nxt)
                    prefetch_bias(nxt)

                wait_rows(slot)

                @pl.when(c + 2 < nchunks)
                def _():
                    fire_idx(c + 2, slot)

                @pl.when(c >= 2)
                def _():
                    wait_out(c - 2, slot)

                compute(c, slot)
                fire_out(c, slot)

        wait_out(nchunks - 2, 0)
        wait_out(nchunks - 1, 1)

    return run


def kernel(gram, context, word_table, context_table, position_weights, bias):
    b, n_neigh, seq = gram.shape
    vocab, embed = word_table.shape
    n_total = b * n_neigh * seq
    assert n_total % (NWORK * CHUNK) == 0

    gram_f = gram.reshape(n_total).astype(jnp.int32)
    ctx_f = context.reshape(n_total).astype(jnp.int32)
    wt_p = lax.bitcast_convert_type(
        word_table.astype(jnp.bfloat16).reshape(vocab, embed // 2, 2), jnp.int32)
    ct_p = lax.bitcast_convert_type(
        context_table.astype(jnp.bfloat16).reshape(vocab, embed // 2, 2), jnp.int32)

    run = _sc_kernel(n_total, vocab, embed, n_neigh, seq)
    out_flat = run(gram_f, ctx_f, wt_p, ct_p,
                   position_weights.astype(jnp.float32), bias.astype(jnp.float32))
    return out_flat.reshape(b, n_neigh, seq)


# speedup vs baseline: 12.6061x; 1.0731x over previous
import functools

import jax
import jax.numpy as jnp
from jax import lax
from jax.experimental import pallas as pl
from jax.experimental.pallas import tpu as pltpu
from jax.experimental.pallas import tpu_sc as plsc

NC = 2
NS = 16
LANES = 16
NWORK = NC * NS
CHUNK = 128


def _sc_kernel(n_total, vocab, embed, n_neigh, seq):
    pairs = embed // 2
    n_w = n_total // NWORK
    nchunks = n_w // CHUNK
    assert nchunks % 2 == 0
    groups = CHUNK // LANES
    mesh = plsc.VectorSubcoreMesh(
        core_axis_name="c", subcore_axis_name="s",
        num_cores=NC, num_subcores=NS)

    @functools.partial(
        pl.kernel,
        out_type=jax.ShapeDtypeStruct((n_total,), jnp.float32),
        mesh=mesh,
        compiler_params=pltpu.CompilerParams(
            needs_layout_passes=False, use_tc_tiling_on_sc=False),
        scratch_types=dict(
            idx_g=pltpu.VMEM((2, CHUNK), jnp.int32),
            idx_c=pltpu.VMEM((2, CHUNK), jnp.int32),
            wrows=pltpu.VMEM((2, CHUNK, pairs), jnp.int32),
            crows=pltpu.VMEM((2, CHUNK, pairs), jnp.int32),
            bv=pltpu.VMEM((2, CHUNK), jnp.float32),
            outv=pltpu.VMEM((2, CHUNK), jnp.float32),
            pw_v=pltpu.VMEM((n_neigh,), jnp.float32),
            bias_v=pltpu.VMEM((vocab,), jnp.float32),
            sem_idx=pltpu.SemaphoreType.DMA((2,)),
            sem_rows=pltpu.SemaphoreType.DMA((2,)),
            sem_out=pltpu.SemaphoreType.DMA((2,)),
        ),
    )
    def run(gram_hbm, ctx_hbm, wt_hbm, ct_hbm, pw_hbm, bias_hbm, out_hbm,
            idx_g, idx_c, wrows, crows, bv, outv, pw_v, bias_v,
            sem_idx, sem_rows, sem_out):
        wid = lax.axis_index("s") * NC + lax.axis_index("c")
        base = wid * n_w
        pltpu.sync_copy(bias_hbm, bias_v)
        pltpu.sync_copy(pw_hbm, pw_v)
        iota16 = lax.iota(jnp.int32, LANES)

        def fire_idx(c, slot):
            st = base + c * CHUNK
            pltpu.async_copy(gram_hbm.at[pl.ds(st, CHUNK)], idx_g.at[slot],
                             sem_idx.at[slot])
            pltpu.async_copy(ctx_hbm.at[pl.ds(st, CHUNK)], idx_c.at[slot],
                             sem_idx.at[slot])

        def wait_idx(c, slot):
            st = base + c * CHUNK
            pltpu.make_async_copy(gram_hbm.at[pl.ds(st, CHUNK)],
                                  idx_g.at[slot], sem_idx.at[slot]).wait()
            pltpu.make_async_copy(ctx_hbm.at[pl.ds(st, CHUNK)],
                                  idx_c.at[slot], sem_idx.at[slot]).wait()

        def fire_rows(slot):
            pltpu.async_copy(wt_hbm.at[idx_g.at[slot]], wrows.at[slot],
                             sem_rows.at[slot])
            pltpu.async_copy(ct_hbm.at[idx_c.at[slot]], crows.at[slot],
                             sem_rows.at[slot])

        def wait_rows(slot):
            pltpu.make_async_copy(wt_hbm.at[idx_g.at[slot]], wrows.at[slot],
                                  sem_rows.at[slot]).wait()
            pltpu.make_async_copy(ct_hbm.at[idx_c.at[slot]], crows.at[slot],
                                  sem_rows.at[slot]).wait()

        def prefetch_bias(slot):
            for g in range(groups):
                idv = idx_c[slot, pl.ds(g * LANES, LANES)]
                bv[slot, pl.ds(g * LANES, LANES)] = plsc.load_gather(
                    bias_v, [idv])

        def fire_out(c, slot):
            st = base + c * CHUNK
            pltpu.async_copy(outv.at[slot], out_hbm.at[pl.ds(st, CHUNK)],
                             sem_out.at[slot])

        def wait_out(c, slot):
            st = base + c * CHUNK
            pltpu.make_async_copy(outv.at[slot], out_hbm.at[pl.ds(st, CHUNK)],
                                  sem_out.at[slot]).wait()

        def compute(c, slot):
            wr = wrows.at[slot]
            cr = crows.at[slot]
            st = base + c * CHUNK
            zero_bf = jnp.zeros((2 * LANES,), jnp.bfloat16)
            for g in range(0, groups, 2):
                rowv0 = iota16 + (g * LANES)
                rowv1 = iota16 + ((g + 1) * LANES)
                acc0 = zero_bf
                acc1 = zero_bf
                for h in range(pairs):
                    colv = jnp.full((LANES,), h, jnp.int32)
                    wv0 = plsc.load_gather(wr, [rowv0, colv])
                    cv0 = plsc.load_gather(cr, [rowv0, colv])
                    wv1 = plsc.load_gather(wr, [rowv1, colv])
                    cv1 = plsc.load_gather(cr, [rowv1, colv])
                    acc0 = acc0 + (plsc.bitcast(wv0, jnp.bfloat16) *
                                   plsc.bitcast(cv0, jnp.bfloat16))
                    acc1 = acc1 + (plsc.bitcast(wv1, jnp.bfloat16) *
                                   plsc.bitcast(cv1, jnp.bfloat16))
                for k, acc in ((g, acc0), (g + 1, acc1)):
                    pa, pb = plsc.unpack(acc,
                                         format=plsc.PackFormat.INTERLEAVED)
                    posv = iota16 + (st + k * LANES)
                    xv = (posv // seq) % n_neigh
                    pwv = plsc.load_gather(pw_v, [xv])
                    bvv = bv[slot, pl.ds(k * LANES, LANES)]
                    outv[slot, pl.ds(k * LANES, LANES)] = \
                        (pa + pb) * pwv + bvv

        fire_idx(0, 0)
        fire_idx(1, 1)
        wait_idx(0, 0)
        fire_rows(0)
        prefetch_bias(0)

        @pl.loop(0, nchunks // 2)
        def chunk_pair(j):
            for b in range(2):
                c = 2 * j + b
                slot = b
                nxt = 1 - b

                @pl.when(c + 1 < nchunks)
                def _():
                    wait_idx(c + 1, nxt)
                    fire_rows(nxt)
                    prefetch_bias(nxt)

                wait_rows(slot)

                @pl.when(c + 2 < nchunks)
                def _():
                    fire_idx(c + 2, slot)

                @pl.when(c >= 2)
                def _():
                    wait_out(c - 2, slot)

                compute(c, slot)
                fire_out(c, slot)

        wait_out(nchunks - 2, 0)
        wait_out(nchunks - 1, 1)

    return run


def kernel(gram, context, word_table, context_table, position_weights, bias):
    b, n_neigh, seq = gram.shape
    vocab, embed = word_table.shape
    n_total = b * n_neigh * seq
    assert n_total % (NWORK * CHUNK) == 0

    gram_f = gram.reshape(n_total).astype(jnp.int32)
    ctx_f = context.reshape(n_total).astype(jnp.int32)
    wt_p = lax.bitcast_convert_type(
        word_table.astype(jnp.bfloat16).reshape(vocab, embed // 2, 2), jnp.int32)
    ct_p = lax.bitcast_convert_type(
        context_table.astype(jnp.bfloat16).reshape(vocab, embed // 2, 2), jnp.int32)

    run = _sc_kernel(n_total, vocab, embed, n_neigh, seq)
    out_flat = run(gram_f, ctx_f, wt_p, ct_p,
                   position_weights.astype(jnp.float32), bias.astype(jnp.float32))
    return out_flat.reshape(b, n_neigh, seq)


# speedup vs baseline: 16.3288x; 1.2953x over previous
import functools

import jax
import jax.numpy as jnp
from jax import lax
from jax.experimental import pallas as pl
from jax.experimental.pallas import tpu as pltpu
from jax.experimental.pallas import tpu_sc as plsc

NC = 2
NS = 16
LANES = 16
NWORK = NC * NS
CHUNK = 128


def _sc_kernel(n_total, vocab, embed, n_neigh, seq):
    pairs = embed // 2
    n_w = n_total // NWORK
    nchunks = n_w // CHUNK
    assert nchunks % 2 == 0
    groups = CHUNK // LANES
    mesh = plsc.VectorSubcoreMesh(
        core_axis_name="c", subcore_axis_name="s",
        num_cores=NC, num_subcores=NS)

    @functools.partial(
        pl.kernel,
        out_type=jax.ShapeDtypeStruct((n_total,), jnp.float32),
        mesh=mesh,
        compiler_params=pltpu.CompilerParams(
            needs_layout_passes=False, use_tc_tiling_on_sc=False),
        scratch_types=dict(
            idx_g=pltpu.VMEM((2, CHUNK), jnp.int32),
            idx_c=pltpu.VMEM((2, CHUNK), jnp.int32),
            wrows=pltpu.VMEM((2, CHUNK, pairs), jnp.int32),
            crows=pltpu.VMEM((2, CHUNK, pairs), jnp.int32),
            bv=pltpu.VMEM((2, CHUNK), jnp.float32),
            outv=pltpu.VMEM((2, CHUNK), jnp.float32),
            pw_v=pltpu.VMEM((n_neigh,), jnp.float32),
            bias_v=pltpu.VMEM((vocab,), jnp.float32),
            sem_idx=pltpu.SemaphoreType.DMA((2,)),
            sem_rows=pltpu.SemaphoreType.DMA((2,)),
            sem_out=pltpu.SemaphoreType.DMA((2,)),
        ),
    )
    def run(gram_hbm, ctx_hbm, wt_hbm, ct_hbm, pw_hbm, bias_hbm, out_hbm,
            idx_g, idx_c, wrows, crows, bv, outv, pw_v, bias_v,
            sem_idx, sem_rows, sem_out):
        wid = lax.axis_index("s") * NC + lax.axis_index("c")
        base = wid * n_w
        pltpu.sync_copy(bias_hbm, bias_v)
        pltpu.sync_copy(pw_hbm, pw_v)
        iota16 = lax.iota(jnp.int32, LANES)

        def fire_idx(c, slot):
            st = base + c * CHUNK
            pltpu.async_copy(gram_hbm.at[pl.ds(st, CHUNK)], idx_g.at[slot],
                             sem_idx.at[slot])
            pltpu.async_copy(ctx_hbm.at[pl.ds(st, CHUNK)], idx_c.at[slot],
                             sem_idx.at[slot])

        def wait_idx(c, slot):
            st = base + c * CHUNK
            pltpu.make_async_copy(gram_hbm.at[pl.ds(st, CHUNK)],
                                  idx_g.at[slot], sem_idx.at[slot]).wait()
            pltpu.make_async_copy(ctx_hbm.at[pl.ds(st, CHUNK)],
                                  idx_c.at[slot], sem_idx.at[slot]).wait()

        def fire_rows(slot):
            pltpu.async_copy(wt_hbm.at[idx_g.at[slot]], wrows.at[slot],
                             sem_rows.at[slot])
            pltpu.async_copy(ct_hbm.at[idx_c.at[slot]], crows.at[slot],
                             sem_rows.at[slot])

        def wait_rows(slot):
            pltpu.make_async_copy(wt_hbm.at[idx_g.at[slot]], wrows.at[slot],
                                  sem_rows.at[slot]).wait()
            pltpu.make_async_copy(ct_hbm.at[idx_c.at[slot]], crows.at[slot],
                                  sem_rows.at[slot]).wait()

        def prefetch_bias(slot):
            for g in range(groups):
                idv = idx_c[slot, pl.ds(g * LANES, LANES)]
                bv[slot, pl.ds(g * LANES, LANES)] = plsc.load_gather(
                    bias_v, [idv])

        def fire_out(c, slot):
            st = base + c * CHUNK
            pltpu.async_copy(outv.at[slot], out_hbm.at[pl.ds(st, CHUNK)],
                             sem_out.at[slot])

        def wait_out(c, slot):
            st = base + c * CHUNK
            pltpu.make_async_copy(outv.at[slot], out_hbm.at[pl.ds(st, CHUNK)],
                                  sem_out.at[slot]).wait()

        def compute(c, slot):
            wr = wrows.at[slot]
            cr = crows.at[slot]
            st = base + c * CHUNK
            zero_bf = jnp.zeros((2 * LANES,), jnp.bfloat16)
            for g in range(0, groups, 2):
                rowv0 = iota16 + (g * LANES)
                rowv1 = iota16 + ((g + 1) * LANES)
                acc0 = zero_bf
                acc1 = zero_bf
                for h in range(pairs):
                    colv = jnp.full((LANES,), h, jnp.int32)
                    wv0 = plsc.load_gather(wr, [rowv0, colv])
                    cv0 = plsc.load_gather(cr, [rowv0, colv])
                    wv1 = plsc.load_gather(wr, [rowv1, colv])
                    cv1 = plsc.load_gather(cr, [rowv1, colv])
                    acc0 = acc0 + (plsc.bitcast(wv0, jnp.bfloat16) *
                                   plsc.bitcast(cv0, jnp.bfloat16))
                    acc1 = acc1 + (plsc.bitcast(wv1, jnp.bfloat16) *
                                   plsc.bitcast(cv1, jnp.bfloat16))
                for k, acc in ((g, acc0), (g + 1, acc1)):
                    pa, pb = plsc.unpack(acc,
                                         format=plsc.PackFormat.INTERLEAVED)
                    posv = iota16 + (st + k * LANES)
                    xv = (posv // seq) % n_neigh
                    pwv = plsc.load_gather(pw_v, [xv])
                    bvv = bv[slot, pl.ds(k * LANES, LANES)]
                    outv[slot, pl.ds(k * LANES, LANES)] = \
                        (pa + pb) * pwv + bvv

        fire_idx(0, 0)
        fire_idx(1, 1)
        wait_idx(0, 0)
        fire_rows(0)
        prefetch_bias(0)

        @pl.loop(0, nchunks // 2)
        def chunk_pair(j):
            for b in range(2):
                c = 2 * j + b
                slot = b
                nxt = 1 - b

                @pl.when(c + 1 < nchunks)
                def _():
                    wait_idx(c + 1, nxt)
                    fire_rows(nxt)
                    prefetch_bias(nxt)

                wait_rows(slot)

                @pl.when(c + 2 < nchunks)
                def _():
                    fire_idx(c + 2, slot)

                @pl.when(c >= 2)
                def _():
                    wait_out(c - 2, slot)

                compute(c, slot)
                fire_out(c, slot)

        wait_out(nchunks - 2, 0)
        wait_out(nchunks - 1, 1)

    return run


def kernel(gram, context, word_table, context_table, position_weights, bias):
    b, n_neigh, seq = gram.shape
    vocab, embed = word_table.shape
    n_total = b * n_neigh * seq
    assert n_total % (NWORK * CHUNK) == 0

    gram_f = gram.reshape(n_total).astype(jnp.int32)
    ctx_f = context.reshape(n_total).astype(jnp.int32)

    def pack_halves(table):
        tb = table.astype(jnp.bfloat16)
        half = embed // 2
        lo = lax.bitcast_convert_type(tb[:, :half], jnp.uint16).astype(jnp.uint32)
        hi = lax.bitcast_convert_type(tb[:, half:], jnp.uint16).astype(jnp.uint32)
        return lax.bitcast_convert_type(lo | (hi << 16), jnp.int32)

    wt_p = pack_halves(word_table)
    ct_p = pack_halves(context_table)

    run = _sc_kernel(n_total, vocab, embed, n_neigh, seq)
    out_flat = run(gram_f, ctx_f, wt_p, ct_p,
                   position_weights.astype(jnp.float32), bias.astype(jnp.float32))
    return out_flat.reshape(b, n_neigh, seq)
